# baseline (speedup 1.0000x reference)
import functools

import jax
import jax.numpy as jnp
from jax import lax
from jax.experimental import pallas as pl
from jax.experimental.pallas import tpu as pltpu
from jax.experimental.pallas import tpu_sc as plsc

_INFO = plsc.get_sparse_core_info()
_NC = _INFO.num_cores
_NS = _INFO.num_subcores
_NL = _INFO.num_lanes
_NW = _NC * _NS

_BATCH = 16384
_DIM = 64
_PDIM = 128
_BPW = _BATCH // _NW
_CHUNK = 128
_NCHUNK = _BPW // _CHUNK

_ENT = 1000000
_FULLC = _ENT // _PDIM
_TAILW = _ENT - _FULLC * _PDIM
_CPW = _FULLC // _NW
_CREM = _FULLC - _CPW * _NW
_RELC = 8
_PITCH = 129


def _transpose_block(win, outb, q0, width16):
    lane = lax.iota(jnp.int32, _NL)

    def row(q, _):
        for h in range(2):
            l = 2 * q + h
            for k in range(4):
                jv = k * _NL + lane
                v = plsc.load_gather(win, [jv, jnp.zeros((_NL,),
                                                         jnp.int32) + l])
                outb[q, pl.ds(h * _DIM + k * _NL, _NL)] = v
        return 0

    lax.fori_loop(0, width16, row, 0)


def _xpose_body(entT, relT, tail2, ent2, rel2, w0, w1, o0, o1,
                sem_i, sem_o):
    wid = lax.axis_index("s") * _NC + lax.axis_index("c")
    wins = (w0, w1)
    outs = (o0, o1)

    extra = jnp.minimum(wid, _CREM)
    c0 = wid * _CPW + extra
    ncols = _CPW + jnp.where(wid < _CREM, 1, 0)

    def fire_in(c, buf):
        pltpu.async_copy(entT.at[:, pl.ds(c * _PDIM, _PDIM)],
                         wins[buf].at[:, pl.ds(0, _PDIM)], sem_i)

    def wait_in():
        pltpu.make_async_copy(entT.at[:, pl.ds(0, _PDIM)],
                              w0.at[:, pl.ds(0, _PDIM)], sem_i).wait()

    def wait_out():
        pltpu.make_async_copy(entT.at[:, pl.ds(0, _PDIM)], o0,
                              sem_o).wait()

    fire_in(c0, 0)

    def step(i2, _):
        for buf in range(2):
            i = i2 * 2 + buf

            @pl.when(i < ncols)
            def _(i=i, buf=buf):
                @pl.when(i + 1 < ncols)
                def _():
                    fire_in(c0 + i + 1, 1 - buf)

                wait_in()

                @pl.when(i >= 2)
                def _():
                    wait_out()

                _transpose_block(wins[buf], outs[buf], 0, _DIM)
                pltpu.async_copy(outs[buf],
                                 ent2.at[pl.ds((c0 + i) * _DIM, _DIM)],
                                 sem_o)
        return 0

    lax.fori_loop(0, (_CPW + 2) // 2, step, 0)

    wait_out()
    wait_out()

    @pl.when(wid == _NW - 1)
    def _():
        pltpu.sync_copy(tail2, o0.at[pl.ds(0, _TAILW // 2)])
        pltpu.sync_copy(o0.at[pl.ds(0, _TAILW // 2)],
                        ent2.at[pl.ds(_FULLC * _DIM, _TAILW // 2)])

    @pl.when(wid == 0)
    def _():
        for c in range(_RELC):
            pltpu.sync_copy(relT.at[:, pl.ds(c * _PDIM, _PDIM)],
                            w0.at[:, pl.ds(0, _PDIM)])
            _transpose_block(w0, o0, 0, _DIM)
            pltpu.sync_copy(o0, rel2.at[pl.ds(c * _DIM, _DIM)])


def _rsqrt(x):
    xi = plsc.bitcast(x, jnp.int32)
    y = plsc.bitcast(jnp.int32(0x5F3759DF) - (xi >> 1), jnp.float32)
    for _ in range(3):
        y = y * (1.5 - 0.5 * x * y * y)
    return y


def _gather_body(bh, bt, br, ent, rel, out, idx_h, idx_t, idx_r,
                 ph, pt, pr, hb, tb, rb, out_v, sem_i, s0, s1):
    wid = lax.axis_index("s") * _NC + lax.axis_index("c")
    base = wid * _BPW

    ci = [pltpu.async_copy(src.at[pl.ds(base, _BPW)], dst, sem_i)
          for src, dst in ((bh, idx_h), (bt, idx_t), (br, idx_r))]
    for cp in ci:
        cp.wait()

    lane = lax.iota(jnp.int32, _NL)
    zf = jnp.zeros((_NL,), jnp.float32)

    def mk_pairs(ti, _):
        s = pl.ds(ti * _NL, _NL)
        ph[s] = idx_h[s] >> 1
        pt[s] = idx_t[s] >> 1
        pr[s] = idx_r[s] >> 1
        return 0

    lax.fori_loop(0, _BPW // _NL, mk_pairs, 0)

    sems = (s0, s1)

    def gather(c, buf):
        rows = pl.ds(c * _CHUNK, _CHUNK)
        return [
            pltpu.async_copy(ent.at[ph.at[rows]], hb.at[buf], sems[buf]),
            pltpu.async_copy(ent.at[pt.at[rows]], tb.at[buf], sems[buf]),
            pltpu.async_copy(rel.at[pr.at[rows]], rb.at[buf], sems[buf]),
        ]

    pend = gather(0, 0)
    for c in range(_NCHUNK):
        for cp in pend:
            cp.wait()
        cur = c % 2
        if c + 1 < _NCHUNK:
            pend = gather(c + 1, 1 - cur)
        hc, tc, rc = hb.at[cur], tb.at[cur], rb.at[cur]

        def group(gi, _, hc=hc, tc=tc, rc=rc, c=c):
            s = pl.ds(c * _CHUNK + gi * _NL, _NL)
            ridx = gi * _NL + lane
            cbh = (idx_h[s] & 1) * _DIM
            cbt = (idx_t[s] & 1) * _DIM
            cbr = (idx_r[s] & 1) * _DIM

            def pass_a(jb, carry):
                hs, rs, ts = carry
                for jo in range(8):
                    cj = (lane + (jb * 8 + jo)) & (_DIM - 1)
                    hj = plsc.load_gather(hc, [ridx, cbh + cj])
                    tj = plsc.load_gather(tc, [ridx, cbt + cj])
                    rj = plsc.load_gather(rc, [ridx, cbr + cj])
                    hs = hs + hj * hj
                    ts = ts + tj * tj
                    rs = rs + rj * rj
                return hs, rs, ts

            hs, rs, ts = lax.fori_loop(0, _DIM // 8, pass_a, (zf, zf, zf))
            ih = _rsqrt(jnp.maximum(hs, 1e-24))
            ir = _rsqrt(jnp.maximum(rs, 1e-24))
            it = _rsqrt(jnp.maximum(ts, 1e-24))

            def pass_b(jb, score):
                for jo in range(8):
                    cj = (lane + (jb * 8 + jo)) & (_DIM - 1)
                    hj = plsc.load_gather(hc, [ridx, cbh + cj])
                    tj = plsc.load_gather(tc, [ridx, cbt + cj])
                    rj = plsc.load_gather(rc, [ridx, cbr + cj])
                    score = score + jnp.abs(hj * ih + rj * ir - tj * it)
                return score

            score = lax.fori_loop(0, _DIM // 8, pass_b, zf)
            out_v[pl.ds(c * _CHUNK + gi * _NL, _NL)] = score
            return 0

        lax.fori_loop(0, _CHUNK // _NL, group, 0)

    pltpu.sync_copy(out_v, out.at[pl.ds(base, _BPW)])


def kernel(batch_h, batch_t, batch_r, ent_emb, rel_emb):
    mesh = plsc.VectorSubcoreMesh(core_axis_name="c", subcore_axis_name="s")
    cp = pltpu.CompilerParams(needs_layout_passes=False,
                              use_tc_tiling_on_sc=True)

    xpose = functools.partial(
        pl.kernel,
        mesh=mesh,
        compiler_params=cp,
        out_type=(jax.ShapeDtypeStruct((_ENT // 2, _PDIM), jnp.float32),
                  jax.ShapeDtypeStruct((_RELC * _DIM, _PDIM), jnp.float32)),
        scratch_types=[
            pltpu.VMEM((_DIM, _PITCH), jnp.float32),
            pltpu.VMEM((_DIM, _PITCH), jnp.float32),
            pltpu.VMEM((_DIM, _PDIM), jnp.float32),
            pltpu.VMEM((_DIM, _PDIM), jnp.float32),
            pltpu.SemaphoreType.DMA,
            pltpu.SemaphoreType.DMA,
        ],
    )(_xpose_body)

    gath = functools.partial(
        pl.kernel,
        mesh=mesh,
        compiler_params=cp,
        out_type=jax.ShapeDtypeStruct((_BATCH,), jnp.float32),
        scratch_types=[
            pltpu.VMEM((_BPW,), jnp.int32),
            pltpu.VMEM((_BPW,), jnp.int32),
            pltpu.VMEM((_BPW,), jnp.int32),
            pltpu.VMEM((_BPW,), jnp.int32),
            pltpu.VMEM((_BPW,), jnp.int32),
            pltpu.VMEM((_BPW,), jnp.int32),
            pltpu.VMEM((2, _CHUNK, _PDIM), jnp.float32),
            pltpu.VMEM((2, _CHUNK, _PDIM), jnp.float32),
            pltpu.VMEM((2, _CHUNK, _PDIM), jnp.float32),
            pltpu.VMEM((_BPW,), jnp.float32),
            pltpu.SemaphoreType.DMA,
            pltpu.SemaphoreType.DMA,
            pltpu.SemaphoreType.DMA,
        ],
    )(_gather_body)

    relT = jnp.pad(rel_emb.T, ((0, 0), (0, _RELC * _PDIM - 1000)))
    tail2 = ent_emb[_FULLC * _PDIM:].reshape(_TAILW // 2, _PDIM)
    ent2, rel2 = xpose(ent_emb.T, relT, tail2)
    return gath(batch_h, batch_t, batch_r, ent2, rel2)


# speedup vs baseline: 2.7207x; 2.7207x over previous
import functools

import jax
import jax.numpy as jnp
from jax import lax
from jax.experimental import pallas as pl
from jax.experimental.pallas import tpu as pltpu
from jax.experimental.pallas import tpu_sc as plsc

_INFO = plsc.get_sparse_core_info()
_NC = _INFO.num_cores
_NS = _INFO.num_subcores
_NL = _INFO.num_lanes
_NW = _NC * _NS

_BATCH = 16384
_DIM = 64
_PDIM = 128
_BPW = _BATCH // _NW
_CHUNK = 128
_NCHUNK = _BPW // _CHUNK

_ENT = 1000000
_FULLC = _ENT // _PDIM
_TAILW = _ENT - _FULLC * _PDIM
_CPW = _FULLC // _NW
_CREM = _FULLC - _CPW * _NW
_RELC = 8


def _transpose_block(win, outb):
    lane = lax.iota(jnp.int32, _NL)
    cols = []
    for b0 in range(8):
        lv = b0 * _NL + lane
        cols.append((lv, lv >> 1, (lv & 1) * _DIM))

    def drow(d, _):
        jv = (d + lane) & (_DIM - 1)
        for lv, rv, bc in cols:
            v = plsc.load_gather(win, [jv, lv])
            plsc.store_scatter(outb, [rv, bc + jv], v)
        return 0

    lax.fori_loop(0, _DIM, drow, 0)


def _xpose_body(entT, relT, tail2, ent2, rel2, w0, w1, o0, o1,
                sem_i, sem_o):
    wid = lax.axis_index("s") * _NC + lax.axis_index("c")
    wins = (w0, w1)
    outs = (o0, o1)

    extra = jnp.minimum(wid, _CREM)
    c0 = wid * _CPW + extra
    ncols = _CPW + jnp.where(wid < _CREM, 1, 0)

    def fire_in(c, buf):
        pltpu.async_copy(entT.at[:, pl.ds(c * _PDIM, _PDIM)],
                         wins[buf].at[:, pl.ds(0, _PDIM)], sem_i)

    def wait_in():
        pltpu.make_async_copy(entT.at[:, pl.ds(0, _PDIM)],
                              w0.at[:, pl.ds(0, _PDIM)], sem_i).wait()

    def wait_out():
        pltpu.make_async_copy(entT.at[:, pl.ds(0, _PDIM)], o0,
                              sem_o).wait()

    fire_in(c0, 0)

    def step(i2, _):
        for buf in range(2):
            i = i2 * 2 + buf

            @pl.when(i < ncols)
            def _(i=i, buf=buf):
                @pl.when(i + 1 < ncols)
                def _():
                    fire_in(c0 + i + 1, 1 - buf)

                wait_in()

                @pl.when(i >= 2)
                def _():
                    wait_out()

                _transpose_block(wins[buf], outs[buf])
                pltpu.async_copy(outs[buf],
                                 ent2.at[pl.ds((c0 + i) * _DIM, _DIM)],
                                 sem_o)
        return 0

    lax.fori_loop(0, (_CPW + 2) // 2, step, 0)

    wait_out()
    wait_out()

    @pl.when(wid == _NW - 1)
    def _():
        pltpu.sync_copy(tail2, o0.at[pl.ds(0, _TAILW // 2)])
        pltpu.sync_copy(o0.at[pl.ds(0, _TAILW // 2)],
                        ent2.at[pl.ds(_FULLC * _DIM, _TAILW // 2)])

    @pl.when(wid == 0)
    def _():
        for c in range(_RELC):
            pltpu.sync_copy(relT.at[:, pl.ds(c * _PDIM, _PDIM)],
                            w0.at[:, pl.ds(0, _PDIM)])
            _transpose_block(w0, o0)
            pltpu.sync_copy(o0, rel2.at[pl.ds(c * _DIM, _DIM)])


def _rsqrt(x):
    xi = plsc.bitcast(x, jnp.int32)
    y = plsc.bitcast(jnp.int32(0x5F3759DF) - (xi >> 1), jnp.float32)
    for _ in range(3):
        y = y * (1.5 - 0.5 * x * y * y)
    return y


def _gather_body(bh, bt, br, ent, rel, out, idx_h, idx_t, idx_r,
                 ph, pt, pr, hb, tb, rb, out_v, sem_i, s0, s1):
    wid = lax.axis_index("s") * _NC + lax.axis_index("c")
    base = wid * _BPW

    ci = [pltpu.async_copy(src.at[pl.ds(base, _BPW)], dst, sem_i)
          for src, dst in ((bh, idx_h), (bt, idx_t), (br, idx_r))]
    for cp in ci:
        cp.wait()

    lane = lax.iota(jnp.int32, _NL)
    zf = jnp.zeros((_NL,), jnp.float32)

    def mk_pairs(ti, _):
        s = pl.ds(ti * _NL, _NL)
        ph[s] = idx_h[s] >> 1
        pt[s] = idx_t[s] >> 1
        pr[s] = idx_r[s] >> 1
        return 0

    lax.fori_loop(0, _BPW // _NL, mk_pairs, 0)

    sems = (s0, s1)

    def gather(c, buf):
        rows = pl.ds(c * _CHUNK, _CHUNK)
        return [
            pltpu.async_copy(ent.at[ph.at[rows]], hb.at[buf], sems[buf]),
            pltpu.async_copy(ent.at[pt.at[rows]], tb.at[buf], sems[buf]),
            pltpu.async_copy(rel.at[pr.at[rows]], rb.at[buf], sems[buf]),
        ]

    pend = gather(0, 0)
    for c in range(_NCHUNK):
        for cp in pend:
            cp.wait()
        cur = c % 2
        if c + 1 < _NCHUNK:
            pend = gather(c + 1, 1 - cur)
        hc, tc, rc = hb.at[cur], tb.at[cur], rb.at[cur]

        def group(gi, _, hc=hc, tc=tc, rc=rc, c=c):
            s = pl.ds(c * _CHUNK + gi * _NL, _NL)
            ridx = gi * _NL + lane
            cbh = (idx_h[s] & 1) * _DIM
            cbt = (idx_t[s] & 1) * _DIM
            cbr = (idx_r[s] & 1) * _DIM

            def pass_a(jb, carry):
                hs, rs, ts = carry
                for jo in range(8):
                    cj = (lane + (jb * 8 + jo)) & (_DIM - 1)
                    hj = plsc.load_gather(hc, [ridx, cbh + cj])
                    tj = plsc.load_gather(tc, [ridx, cbt + cj])
                    rj = plsc.load_gather(rc, [ridx, cbr + cj])
                    hs = hs + hj * hj
                    ts = ts + tj * tj
                    rs = rs + rj * rj
                return hs, rs, ts

            hs, rs, ts = lax.fori_loop(0, _DIM // 8, pass_a, (zf, zf, zf))
            ih = _rsqrt(jnp.maximum(hs, 1e-24))
            ir = _rsqrt(jnp.maximum(rs, 1e-24))
            it = _rsqrt(jnp.maximum(ts, 1e-24))

            def pass_b(jb, score):
                for jo in range(8):
                    cj = (lane + (jb * 8 + jo)) & (_DIM - 1)
                    hj = plsc.load_gather(hc, [ridx, cbh + cj])
                    tj = plsc.load_gather(tc, [ridx, cbt + cj])
                    rj = plsc.load_gather(rc, [ridx, cbr + cj])
                    score = score + jnp.abs(hj * ih + rj * ir - tj * it)
                return score

            score = lax.fori_loop(0, _DIM // 8, pass_b, zf)
            out_v[pl.ds(c * _CHUNK + gi * _NL, _NL)] = score
            return 0

        lax.fori_loop(0, _CHUNK // _NL, group, 0)

    pltpu.sync_copy(out_v, out.at[pl.ds(base, _BPW)])


def kernel(batch_h, batch_t, batch_r, ent_emb, rel_emb):
    mesh = plsc.VectorSubcoreMesh(core_axis_name="c", subcore_axis_name="s")
    cp = pltpu.CompilerParams(needs_layout_passes=False,
                              use_tc_tiling_on_sc=True)

    xpose = functools.partial(
        pl.kernel,
        mesh=mesh,
        compiler_params=cp,
        out_type=(jax.ShapeDtypeStruct((_ENT // 2, _PDIM), jnp.float32),
                  jax.ShapeDtypeStruct((_RELC * _DIM, _PDIM), jnp.float32)),
        scratch_types=[
            pltpu.VMEM((_DIM, _PDIM), jnp.float32),
            pltpu.VMEM((_DIM, _PDIM), jnp.float32),
            pltpu.VMEM((_DIM, _PDIM), jnp.float32),
            pltpu.VMEM((_DIM, _PDIM), jnp.float32),
            pltpu.SemaphoreType.DMA,
            pltpu.SemaphoreType.DMA,
        ],
    )(_xpose_body)

    gath = functools.partial(
        pl.kernel,
        mesh=mesh,
        compiler_params=cp,
        out_type=jax.ShapeDtypeStruct((_BATCH,), jnp.float32),
        scratch_types=[
            pltpu.VMEM((_BPW,), jnp.int32),
            pltpu.VMEM((_BPW,), jnp.int32),
            pltpu.VMEM((_BPW,), jnp.int32),
            pltpu.VMEM((_BPW,), jnp.int32),
            pltpu.VMEM((_BPW,), jnp.int32),
            pltpu.VMEM((_BPW,), jnp.int32),
            pltpu.VMEM((2, _CHUNK, _PDIM), jnp.float32),
            pltpu.VMEM((2, _CHUNK, _PDIM), jnp.float32),
            pltpu.VMEM((2, _CHUNK, _PDIM), jnp.float32),
            pltpu.VMEM((_BPW,), jnp.float32),
            pltpu.SemaphoreType.DMA,
            pltpu.SemaphoreType.DMA,
            pltpu.SemaphoreType.DMA,
        ],
    )(_gather_body)

    relT = jnp.pad(rel_emb.T, ((0, 0), (0, _RELC * _PDIM - 1000)))
    tail2 = ent_emb[_FULLC * _PDIM:].reshape(_TAILW // 2, _PDIM)
    ent2, rel2 = xpose(ent_emb.T, relT, tail2)
    return gath(batch_h, batch_t, batch_r, ent2, rel2)
